# baseline (speedup 1.0000x reference)
import jax
import jax.numpy as jnp
from jax.experimental import pallas as pl
from jax.experimental.pallas import tpu as pltpu

_MS = 8
_CLASSES = 5
_BH = 32


def _fused_body(wb_ref, x_ref, lab_ref, o_ref):
    bh, wdim = lab_ref.shape[2], lab_ref.shape[3]
    pb = bh * wdim
    xb = x_ref[0].astype(jnp.bfloat16)
    x = xb.reshape(x_ref.shape[1], pb)
    x1 = jnp.concatenate([x, jnp.ones((1, pb), jnp.bfloat16)], axis=0)
    wb = wb_ref[...].astype(jnp.bfloat16)
    y = jnp.dot(wb, x1, preferred_element_type=jnp.float32)
    lab = lab_ref[0].reshape(1, pb)
    acc = y[0:_MS]
    for l in range(1, _CLASSES):
        acc = jnp.where(lab == l, y[l * _MS:(l + 1) * _MS], acc)
    o_ref[0] = jnp.maximum(acc, 0.0).reshape(_MS, bh, wdim)


def kernel(image, clusters, W0, b0, W1, b1, W2, b2, W3, b3, W4, b4):
    Bb, C, Hh, Ww = image.shape
    nb = Hh // _BH
    wcat = jnp.concatenate([W0, W1, W2, W3, W4], axis=0)
    bcat = jnp.concatenate([b0, b1, b2, b3, b4], axis=0)[:, None]
    wbcat = jnp.concatenate([wcat, bcat], axis=1)

    return pl.pallas_call(
        _fused_body,
        grid=(Bb, nb),
        in_specs=[
            pl.BlockSpec((_CLASSES * _MS, C + 1), lambda b, j: (0, 0)),
            pl.BlockSpec((1, C, _BH, Ww), lambda b, j: (b, 0, j, 0)),
            pl.BlockSpec((1, 1, _BH, Ww), lambda b, j: (b, 0, j, 0)),
        ],
        out_specs=pl.BlockSpec((1, _MS, _BH, Ww), lambda b, j: (b, 0, j, 0)),
        out_shape=jax.ShapeDtypeStruct((Bb, _MS, Hh, Ww), jnp.float32),
        compiler_params=pltpu.CompilerParams(
            dimension_semantics=("arbitrary", "arbitrary"),
        ),
    )(wbcat, image, clusters)


# speedup vs baseline: 1.0967x; 1.0967x over previous
import jax
import jax.numpy as jnp
from jax.experimental import pallas as pl
from jax.experimental.pallas import tpu as pltpu

_MS = 8
_CLASSES = 5
_BH = 128


def _fused_body(wb_ref, x_ref, lab_ref, o_ref):
    bh, wdim = lab_ref.shape[2], lab_ref.shape[3]
    pb = bh * wdim
    xb = x_ref[0].astype(jnp.bfloat16)
    x = xb.reshape(x_ref.shape[1], pb)
    x1 = jnp.concatenate([x, jnp.ones((1, pb), jnp.bfloat16)], axis=0)
    wb = wb_ref[...].astype(jnp.bfloat16)
    y = jnp.dot(wb, x1, preferred_element_type=jnp.float32)
    lab = lab_ref[0].reshape(1, pb)
    acc = y[0:_MS]
    for l in range(1, _CLASSES):
        acc = jnp.where(lab == l, y[l * _MS:(l + 1) * _MS], acc)
    o_ref[0] = jnp.maximum(acc, 0.0).reshape(_MS, bh, wdim)


def kernel(image, clusters, W0, b0, W1, b1, W2, b2, W3, b3, W4, b4):
    Bb, C, Hh, Ww = image.shape
    nb = Hh // _BH
    wcat = jnp.concatenate([W0, W1, W2, W3, W4], axis=0)
    bcat = jnp.concatenate([b0, b1, b2, b3, b4], axis=0)[:, None]
    wbcat = jnp.concatenate([wcat, bcat], axis=1)

    return pl.pallas_call(
        _fused_body,
        grid=(Bb, nb),
        in_specs=[
            pl.BlockSpec((_CLASSES * _MS, C + 1), lambda b, j: (0, 0)),
            pl.BlockSpec((1, C, _BH, Ww), lambda b, j: (b, 0, j, 0)),
            pl.BlockSpec((1, 1, _BH, Ww), lambda b, j: (b, 0, j, 0)),
        ],
        out_specs=pl.BlockSpec((1, _MS, _BH, Ww), lambda b, j: (b, 0, j, 0)),
        out_shape=jax.ShapeDtypeStruct((Bb, _MS, Hh, Ww), jnp.float32),
        compiler_params=pltpu.CompilerParams(
            dimension_semantics=("parallel", "parallel"),
        ),
    )(wbcat, image, clusters)
